# baseline (speedup 1.0000x reference)
import functools

import jax
import jax.numpy as jnp
from jax import lax
from jax.experimental import pallas as pl
from jax.experimental.pallas import tpu as pltpu
from jax.experimental.pallas import tpu_sc as plsc

N = 10000
NPAD = 10240
D = 128
E = 320000
NW = 32
EPW = 10240
EPAD = NW * EPW
DWIN = 128
DNWIN = EPW // DWIN
WIN = 128
NBUF = 2
NWIN = EPW // WIN
NHALF = 2
HW = NWIN // NHALF
RPW = NPAD // NW
G = 64
OUT = 8

_MESH = plsc.VectorSubcoreMesh(core_axis_name="c", subcore_axis_name="s")


@functools.partial(
    pl.kernel,
    out_type=jax.ShapeDtypeStruct((2 * NPAD, 16), jnp.float32),
    mesh=_MESH,
    scratch_types=[
        pltpu.VMEM((DNWIN, DWIN), jnp.int32),
        pltpu.VMEM((DWIN, 16), jnp.float32),
        pltpu.VMEM_SHARED((NPAD, 16), jnp.float32),
        pltpu.SemaphoreType.DMA,
    ],
)
def _sc_degree(dst_hbm, ones_hbm, zeros_hbm, deg_hbm, dst_all, ones_v,
               deg_sh, sem):
    c = lax.axis_index("c")
    s = lax.axis_index("s")
    wid = c * 16 + s
    pltpu.sync_copy(ones_hbm, ones_v)
    pltpu.sync_copy(zeros_hbm, deg_sh.at[pl.ds(s * RPW, RPW)])
    pltpu.sync_copy(dst_hbm.at[pl.ds(wid * DNWIN, DNWIN)], dst_all)
    plsc.subcore_barrier()

    @pl.loop(0, DNWIN, step=8)
    def _(w):
        for j in range(8):
            pltpu.async_copy(ones_v, deg_sh.at[dst_all.at[w + j]], sem,
                             add=True)
        for j in range(8):
            pltpu.make_async_copy(ones_v, deg_sh.at[dst_all.at[w + j]],
                                  sem).wait()

    plsc.subcore_barrier()
    pltpu.sync_copy(
        deg_sh.at[pl.ds(s * RPW, RPW)],
        deg_hbm.at[pl.ds(c * NPAD + s * RPW, RPW)],
    )


@functools.partial(
    pl.kernel,
    out_type=jax.ShapeDtypeStruct((2 * NPAD, D), jnp.float32),
    mesh=_MESH,
    scratch_types=[
        pltpu.VMEM((HW, WIN), jnp.int32),
        pltpu.VMEM((HW, WIN), jnp.int32),
        [pltpu.VMEM((WIN, D), jnp.float32)] * NBUF,
        [pltpu.SemaphoreType.DMA] * NBUF,
        [pltpu.SemaphoreType.DMA] * NBUF,
        pltpu.VMEM_SHARED((NPAD, D), jnp.float32),
    ],
)
def _sc_aggregate(h_hbm, src_hbm, dst_hbm, zeros_hbm, agg_hbm,
                  src_h, dst_h, rows, sem_g, sem_s, acc_sh):
    c = lax.axis_index("c")
    s = lax.axis_index("s")
    wid = c * 16 + s
    pltpu.sync_copy(zeros_hbm, acc_sh.at[pl.ds(s * RPW, RPW)])
    plsc.subcore_barrier()

    @pl.loop(0, NHALF)
    def _(half):
        base = wid * NWIN + half * HW
        pltpu.sync_copy(src_hbm.at[pl.ds(base, HW)], src_h)
        pltpu.sync_copy(dst_hbm.at[pl.ds(base, HW)], dst_h)
        for j in range(NBUF):
            pltpu.async_copy(h_hbm.at[src_h.at[j]], rows[j], sem_g[j])

        @pl.loop(0, HW, step=NBUF)
        def _(w):
            descs = []
            for j in range(NBUF):
                pltpu.make_async_copy(h_hbm.at[src_h.at[w + j]], rows[j],
                                      sem_g[j]).wait()
                descs.append(
                    pltpu.async_copy(rows[j], acc_sh.at[dst_h.at[w + j]],
                                     sem_s[j], add=True))
            for j in range(NBUF):
                descs[j].wait()

                @pl.when(w + NBUF + j < HW)
                def _(j=j):
                    pltpu.async_copy(h_hbm.at[src_h.at[w + NBUF + j]],
                                     rows[j], sem_g[j])

    plsc.subcore_barrier()
    pltpu.sync_copy(
        acc_sh.at[pl.ds(s * RPW, RPW)],
        agg_hbm.at[pl.ds(c * NPAD + s * RPW, RPW)],
    )


def _tc_matmul_block(x_ref, w_ref, o_ref):
    o_ref[...] = jnp.dot(x_ref[...], w_ref[...],
                         preferred_element_type=jnp.float32)


def _tc_scale_block(h_ref, d0_ref, d1_ref, o_ref):
    deg = d0_ref[:, :1] + d1_ref[:, :1] + 1.0
    o_ref[...] = h_ref[...] * lax.rsqrt(deg)


def _tc_post_body(agg_ref, hs_ref, deg_ref, bconv_ref, batch_ref,
                  wlin_ref, blin_ref, logits_ref, xpool_ref, hout_ref):
    agg = agg_ref[pl.ds(0, NPAD), :] + agg_ref[pl.ds(NPAD, NPAD), :]
    deg = deg_ref[pl.ds(0, NPAD), :1] + deg_ref[pl.ds(NPAD, NPAD), :1] + 1.0
    dis = lax.rsqrt(deg)
    pre = dis * (agg + hs_ref[...]) + bconv_ref[...]
    hout_ref[...] = jnp.maximum(pre, 0.0)
    b2d = batch_ref[...]

    def graph_body(g, carry):
        start = jnp.sum(jnp.where(b2d < g, 1, 0))
        cnt = jnp.sum(jnp.where(b2d == g, 1, 0))

        def cond(kc):
            return kc[0] * 32 < cnt

        def body(kc):
            k, acc = kc
            rows = hout_ref[pl.ds(start + k * 32, 32), :]
            rid = lax.broadcasted_iota(jnp.int32, (32, D), 0) + k * 32
            rows = jnp.where(rid < cnt, rows, 0.0)
            return k + 1, jnp.maximum(acc, rows)

        _, acc = lax.while_loop(
            cond, body, (jnp.int32(0), jnp.zeros((32, D), jnp.float32)))
        xpool_ref[pl.ds(g, 1), :] = jnp.max(acc, axis=0, keepdims=True)
        return carry

    lax.fori_loop(0, G, graph_body, 0)
    logits_ref[...] = (
        jnp.dot(xpool_ref[...], wlin_ref[...],
                preferred_element_type=jnp.float32) + blin_ref[...])


def kernel(x, edge_index, batch, W_conv, b_conv, W_lin, b_lin):
    x_pad = jnp.zeros((NPAD, D), jnp.float32).at[:N].set(x)
    pad_idx = jnp.full((EPAD - E,), NPAD - 1, jnp.int32)
    src = jnp.concatenate([edge_index[0], pad_idx]).reshape(NW * NWIN, WIN)
    dst_flat = jnp.concatenate([edge_index[1], pad_idx])
    dst = dst_flat.reshape(NW * NWIN, WIN)
    dst_deg = dst_flat.reshape(NW * DNWIN, DWIN)
    ones16 = jnp.ones((DWIN, 16), jnp.float32)
    zeros16 = jnp.zeros((RPW, 16), jnp.float32)
    zerosD = jnp.zeros((RPW, D), jnp.float32)
    batch_pad = jnp.concatenate(
        [batch, jnp.full((NPAD - N,), G, jnp.int32)]).reshape(NPAD // D, D)

    degp = _sc_degree(dst_deg, ones16, zeros16)

    h = pl.pallas_call(
        _tc_matmul_block,
        grid=(NPAD // 256,),
        in_specs=[pl.BlockSpec((256, D), lambda i: (i, 0)),
                  pl.BlockSpec((D, D), lambda i: (0, 0))],
        out_specs=pl.BlockSpec((256, D), lambda i: (i, 0)),
        out_shape=jax.ShapeDtypeStruct((NPAD, D), jnp.float32),
    )(x_pad, W_conv)

    h_scaled = pl.pallas_call(
        _tc_scale_block,
        grid=(NPAD // 256,),
        in_specs=[pl.BlockSpec((256, D), lambda i: (i, 0)),
                  pl.BlockSpec((256, 16), lambda i: (i, 0)),
                  pl.BlockSpec((256, 16), lambda i: (i + NPAD // 256, 0))],
        out_specs=pl.BlockSpec((256, D), lambda i: (i, 0)),
        out_shape=jax.ShapeDtypeStruct((NPAD, D), jnp.float32),
    )(h, degp, degp)

    aggp = _sc_aggregate(h_scaled, src, dst, zerosD)

    logits, x_pool = pl.pallas_call(
        _tc_post_body,
        out_shape=(jax.ShapeDtypeStruct((G, OUT), jnp.float32),
                   jax.ShapeDtypeStruct((G, D), jnp.float32)),
        scratch_shapes=[pltpu.VMEM((NPAD, D), jnp.float32)],
    )(aggp, h_scaled, degp, b_conv.reshape(1, D), batch_pad,
      W_lin, b_lin.reshape(1, OUT))
    return (logits, x_pool)


# speedup vs baseline: 1.0753x; 1.0753x over previous
import functools

import jax
import jax.numpy as jnp
from jax import lax
from jax.experimental import pallas as pl
from jax.experimental.pallas import tpu as pltpu
from jax.experimental.pallas import tpu_sc as plsc

N = 10000
NPAD = 10240
D = 128
E = 320000
NW = 32
EPW = 10240
EPAD = NW * EPW
DWIN = 128
DNWIN = EPW // DWIN
WIN = 128
NBUF = 2
TWIN = EPAD // WIN
CH = 40
W0 = 120
W1 = TWIN // 16 - W0
RPW = NPAD // NW
G = 64
OUT = 8

_MESH = plsc.VectorSubcoreMesh(core_axis_name="c", subcore_axis_name="s")


@functools.partial(
    pl.kernel,
    out_type=jax.ShapeDtypeStruct((2 * NPAD, 16), jnp.float32),
    mesh=_MESH,
    scratch_types=[
        pltpu.VMEM((DNWIN, DWIN), jnp.int32),
        pltpu.VMEM((DWIN, 16), jnp.float32),
        pltpu.VMEM_SHARED((NPAD, 16), jnp.float32),
        pltpu.SemaphoreType.DMA,
    ],
)
def _sc_degree(dst_hbm, ones_hbm, zeros_hbm, deg_hbm, dst_all, ones_v,
               deg_sh, sem):
    c = lax.axis_index("c")
    s = lax.axis_index("s")
    wid = c * 16 + s
    pltpu.sync_copy(ones_hbm, ones_v)
    pltpu.sync_copy(zeros_hbm, deg_sh.at[pl.ds(s * RPW, RPW)])
    pltpu.sync_copy(dst_hbm.at[pl.ds(wid * DNWIN, DNWIN)], dst_all)
    plsc.subcore_barrier()

    @pl.loop(0, DNWIN, step=8)
    def _(w):
        for j in range(8):
            pltpu.async_copy(ones_v, deg_sh.at[dst_all.at[w + j]], sem,
                             add=True)
        for j in range(8):
            pltpu.make_async_copy(ones_v, deg_sh.at[dst_all.at[w + j]],
                                  sem).wait()

    plsc.subcore_barrier()
    pltpu.sync_copy(
        deg_sh.at[pl.ds(s * RPW, RPW)],
        deg_hbm.at[pl.ds(c * NPAD + s * RPW, RPW)],
    )


@functools.partial(
    pl.kernel,
    out_type=jax.ShapeDtypeStruct((2 * NPAD, D), jnp.float32),
    mesh=_MESH,
    scratch_types=[
        pltpu.VMEM((CH, WIN), jnp.int32),
        pltpu.VMEM((CH, WIN), jnp.int32),
        [pltpu.VMEM((WIN, D), jnp.float32)] * NBUF,
        [pltpu.SemaphoreType.DMA] * NBUF,
        [pltpu.SemaphoreType.DMA] * NBUF,
        pltpu.VMEM_SHARED((NPAD, D), jnp.float32),
    ],
)
def _sc_aggregate(h_hbm, src_hbm, dst_hbm, zeros_hbm, agg_hbm,
                  src_h, dst_h, rows, sem_g, sem_s, acc_sh):
    c = lax.axis_index("c")
    s = lax.axis_index("s")
    pltpu.sync_copy(zeros_hbm, acc_sh.at[pl.ds(s * RPW, RPW)])
    plsc.subcore_barrier()
    tile_base = jnp.where(c == 0, s * W0, 16 * W0 + s * W1)
    nch = jnp.where(c == 0, W0 // CH, W1 // CH)

    @pl.loop(0, nch)
    def _(chunk):
        base = tile_base + chunk * CH
        pltpu.sync_copy(src_hbm.at[pl.ds(base, CH)], src_h)
        pltpu.sync_copy(dst_hbm.at[pl.ds(base, CH)], dst_h)
        for j in range(NBUF):
            pltpu.async_copy(h_hbm.at[src_h.at[j]], rows[j], sem_g[j])

        @pl.loop(0, CH, step=NBUF)
        def _(w):
            descs = []
            for j in range(NBUF):
                pltpu.make_async_copy(h_hbm.at[src_h.at[w + j]], rows[j],
                                      sem_g[j]).wait()
                descs.append(
                    pltpu.async_copy(rows[j], acc_sh.at[dst_h.at[w + j]],
                                     sem_s[j], add=True))
            for j in range(NBUF):
                descs[j].wait()

                @pl.when(w + NBUF + j < CH)
                def _(j=j):
                    pltpu.async_copy(h_hbm.at[src_h.at[w + NBUF + j]],
                                     rows[j], sem_g[j])

    plsc.subcore_barrier()
    pltpu.sync_copy(
        acc_sh.at[pl.ds(s * RPW, RPW)],
        agg_hbm.at[pl.ds(c * NPAD + s * RPW, RPW)],
    )


def _tc_matmul_block(x_ref, w_ref, o_ref):
    o_ref[...] = jnp.dot(x_ref[...], w_ref[...],
                         preferred_element_type=jnp.float32)


def _tc_scale_block(h_ref, d0_ref, d1_ref, o_ref):
    deg = d0_ref[:, :1] + d1_ref[:, :1] + 1.0
    o_ref[...] = h_ref[...] * lax.rsqrt(deg)


def _tc_post_body(agg_ref, hs_ref, deg_ref, bconv_ref, batch_ref,
                  wlin_ref, blin_ref, logits_ref, xpool_ref, hout_ref):
    agg = agg_ref[pl.ds(0, NPAD), :] + agg_ref[pl.ds(NPAD, NPAD), :]
    deg = deg_ref[pl.ds(0, NPAD), :1] + deg_ref[pl.ds(NPAD, NPAD), :1] + 1.0
    dis = lax.rsqrt(deg)
    pre = dis * (agg + hs_ref[...]) + bconv_ref[...]
    hout_ref[...] = jnp.maximum(pre, 0.0)
    b2d = batch_ref[...]

    def graph_body(g, carry):
        start = jnp.sum(jnp.where(b2d < g, 1, 0))
        cnt = jnp.sum(jnp.where(b2d == g, 1, 0))

        def cond(kc):
            return kc[0] * 32 < cnt

        def body(kc):
            k, acc = kc
            rows = hout_ref[pl.ds(start + k * 32, 32), :]
            rid = lax.broadcasted_iota(jnp.int32, (32, D), 0) + k * 32
            rows = jnp.where(rid < cnt, rows, 0.0)
            return k + 1, jnp.maximum(acc, rows)

        _, acc = lax.while_loop(
            cond, body, (jnp.int32(0), jnp.zeros((32, D), jnp.float32)))
        xpool_ref[pl.ds(g, 1), :] = jnp.max(acc, axis=0, keepdims=True)
        return carry

    lax.fori_loop(0, G, graph_body, 0)
    logits_ref[...] = (
        jnp.dot(xpool_ref[...], wlin_ref[...],
                preferred_element_type=jnp.float32) + blin_ref[...])


def kernel(x, edge_index, batch, W_conv, b_conv, W_lin, b_lin):
    x_pad = jnp.zeros((NPAD, D), jnp.float32).at[:N].set(x)
    pad_idx = jnp.full((EPAD - E,), NPAD - 1, jnp.int32)
    src = jnp.concatenate([edge_index[0], pad_idx]).reshape(TWIN, WIN)
    dst_flat = jnp.concatenate([edge_index[1], pad_idx])
    dst = dst_flat.reshape(TWIN, WIN)
    dst_deg = dst_flat.reshape(NW * DNWIN, DWIN)
    ones16 = jnp.ones((DWIN, 16), jnp.float32)
    zeros16 = jnp.zeros((RPW, 16), jnp.float32)
    zerosD = jnp.zeros((RPW, D), jnp.float32)
    batch_pad = jnp.concatenate(
        [batch, jnp.full((NPAD - N,), G, jnp.int32)]).reshape(NPAD // D, D)

    degp = _sc_degree(dst_deg, ones16, zeros16)

    h = pl.pallas_call(
        _tc_matmul_block,
        grid=(NPAD // 256,),
        in_specs=[pl.BlockSpec((256, D), lambda i: (i, 0)),
                  pl.BlockSpec((D, D), lambda i: (0, 0))],
        out_specs=pl.BlockSpec((256, D), lambda i: (i, 0)),
        out_shape=jax.ShapeDtypeStruct((NPAD, D), jnp.float32),
    )(x_pad, W_conv)

    h_scaled = pl.pallas_call(
        _tc_scale_block,
        grid=(NPAD // 256,),
        in_specs=[pl.BlockSpec((256, D), lambda i: (i, 0)),
                  pl.BlockSpec((256, 16), lambda i: (i, 0)),
                  pl.BlockSpec((256, 16), lambda i: (i + NPAD // 256, 0))],
        out_specs=pl.BlockSpec((256, D), lambda i: (i, 0)),
        out_shape=jax.ShapeDtypeStruct((NPAD, D), jnp.float32),
    )(h, degp, degp)

    aggp = _sc_aggregate(h_scaled, src, dst, zerosD)

    logits, x_pool = pl.pallas_call(
        _tc_post_body,
        out_shape=(jax.ShapeDtypeStruct((G, OUT), jnp.float32),
                   jax.ShapeDtypeStruct((G, D), jnp.float32)),
        scratch_shapes=[pltpu.VMEM((NPAD, D), jnp.float32)],
    )(aggp, h_scaled, degp, b_conv.reshape(1, D), batch_pad,
      W_lin, b_lin.reshape(1, OUT))
    return (logits, x_pool)
